# baseline (speedup 1.0000x reference)
import jax
import jax.numpy as jnp
from jax.experimental import pallas as pl

N = 4096
BR = 512


def _rowsum_body(data_ref, adj_ref, out_ref):
    out_ref[...] = jnp.sum(adj_ref[...] * data_ref[...], axis=1)


def _softmax_body(x_ref, out_ref):
    x = x_ref[...]
    m = jnp.max(x)
    e = jnp.exp(x - m)
    out_ref[...] = e / jnp.sum(e)


def kernel(data_input, adj_matrix):
    sums = pl.pallas_call(
        _rowsum_body,
        grid=(N // BR,),
        in_specs=[
            pl.BlockSpec((BR, N), lambda i: (i, 0)),
            pl.BlockSpec((BR, N), lambda i: (i, 0)),
        ],
        out_specs=pl.BlockSpec((BR,), lambda i: (i,)),
        out_shape=jax.ShapeDtypeStruct((N,), jnp.float32),
    )(data_input, adj_matrix)

    x2 = sums.reshape(8, N // 8)
    alpha = pl.pallas_call(
        _softmax_body,
        out_shape=jax.ShapeDtypeStruct((8, N // 8), jnp.float32),
    )(x2)
    return alpha.reshape(N)


# speedup vs baseline: 1.0596x; 1.0596x over previous
import jax
import jax.numpy as jnp
from jax.experimental import pallas as pl
from jax.experimental.pallas import tpu as pltpu

N = 4096
BC = 512
GRID = N // BC


def _body(data_ref, adj_ref, out_ref, acc_ref):
    i = pl.program_id(0)
    part = jnp.sum(adj_ref[...] * data_ref[...], axis=1)

    @pl.when(i == 0)
    def _init():
        acc_ref[...] = part

    @pl.when(i > 0)
    def _acc():
        acc_ref[...] += part

    @pl.when(i == GRID - 1)
    def _final():
        x = acc_ref[...]
        m = jnp.max(x)
        e = jnp.exp(x - m)
        out_ref[...] = e / jnp.sum(e)


def kernel(data_input, adj_matrix):
    return pl.pallas_call(
        _body,
        grid=(GRID,),
        in_specs=[
            pl.BlockSpec((N, BC), lambda i: (0, i)),
            pl.BlockSpec((N, BC), lambda i: (0, i)),
        ],
        out_specs=pl.BlockSpec((N,), lambda i: (0,)),
        out_shape=jax.ShapeDtypeStruct((N,), jnp.float32),
        scratch_shapes=[pltpu.VMEM((N,), jnp.float32)],
    )(data_input, adj_matrix)
